# baseline (speedup 1.0000x reference)
import functools

import jax
import jax.numpy as jnp
from jax import lax
from jax.experimental import pallas as pl
from jax.experimental.pallas import tpu as pltpu
from jax.experimental.pallas import tpu_sc as plsc

_LANES = 16
_NC, _NS = 2, 16
_NW = _NC * _NS


def _relayout(word_table):
    V, E = word_table.shape
    TC = V // 128
    TAIL = V - TC * 128
    base_per_w = TC // _NW
    rem = TC - base_per_w * _NW
    per_w = base_per_w + (1 if rem else 0)
    SLOT = 128 * E

    wtT = word_table.T
    tail64 = word_table[TC * 128:]

    @functools.partial(
        pl.kernel,
        mesh=plsc.VectorSubcoreMesh(core_axis_name="c", subcore_axis_name="s"),
        out_type=jax.ShapeDtypeStruct((V * E,), jnp.float32),
        compiler_params=pltpu.CompilerParams(
            use_tc_tiling_on_sc=True, needs_layout_passes=False),
        scratch_types=[
            pltpu.VMEM((E, 128), jnp.float32),
            pltpu.VMEM((E, 128), jnp.float32),
            pltpu.VMEM((128 * 64,), jnp.float32),
            pltpu.VMEM((128 * 64,), jnp.float32),
            pltpu.VMEM((64, 64), jnp.float32),
            pltpu.SemaphoreType.DMA,
            pltpu.SemaphoreType.DMA,
            pltpu.SemaphoreType.DMA,
            pltpu.SemaphoreType.DMA,
        ],
    )
    def relayout_kernel(wtT_hbm, tail_hbm, out_hbm, tiles0, tiles1,
                        rows0, rows1, tv, sr0, sr1, sw0, sw1):
        sem_r = [sr0, sr1]
        sem_w = [sw0, sw1]
        tiles = [tiles0, tiles1]
        rows = [rows0, rows1]
        wid = lax.axis_index("s") * _NC + lax.axis_index("c")
        start = wid * base_per_w + jnp.minimum(wid, rem)
        n_slots = per_w

        def slot(i):
            return jnp.minimum(start + i, TC - 1)

        def read(i, b):
            for c8 in range(E // 8):
                pltpu.async_copy(
                    wtT_hbm.at[pl.ds(8 * c8, 8), pl.ds(slot(i) * 128, 128)],
                    tiles[b].at[pl.ds(8 * c8, 8)], sem_r[b])

        def wait_read(i, b):
            for c8 in range(E // 8):
                pltpu.make_async_copy(
                    wtT_hbm.at[pl.ds(8 * c8, 8), pl.ds(slot(i) * 128, 128)],
                    tiles[b].at[pl.ds(8 * c8, 8)], sem_r[b]).wait()

        def write(i, b):
            pltpu.async_copy(rows[b], out_hbm.at[pl.ds(slot(i) * SLOT, SLOT)],
                             sem_w[b])

        def wait_write(i, b):
            pltpu.make_async_copy(rows[b],
                                  out_hbm.at[pl.ds(slot(i) * SLOT, SLOT)],
                                  sem_w[b]).wait()

        lanes = lax.iota(jnp.int32, _LANES)
        rot = [(lanes + k) % _LANES for k in range(_LANES)]
        d64 = [lanes * E + rot[k] for k in range(_LANES)]

        def transpose(b):
            def body(i, carry):
                row_idx = lanes + i * _LANES
                for c0 in range(0, E, _LANES):
                    dbase = i * (_LANES * E) + c0
                    vals = [plsc.load_gather(tiles[b], [rot[k] + c0, row_idx])
                            for k in range(_LANES)]
                    for k in range(_LANES):
                        plsc.store_scatter(rows[b], [d64[k] + dbase], vals[k])
                return carry
            lax.fori_loop(0, 128 // _LANES, body, 0)

        read(0, 0)

        def step(s, carry):
            for b in range(2):
                i = s * 2 + b
                pl.when(i + 1 < n_slots)(lambda: read(i + 1, 1 - b))
                wait_read(i, b)
                pl.when(i >= 2)(lambda: wait_write(i - 2, b))
                transpose(b)
                write(i, b)
            return carry

        lax.fori_loop(0, n_slots // 2, step, 0)
        if n_slots % 2:
            i = n_slots - 1
            b = i % 2
            wait_read(i, b)
            pl.when(i >= 2)(lambda: wait_write(i - 2, b))
            transpose(b)
            write(i, b)
        for b in range(2):
            i = n_slots - 2 + b
            wait_write(i, i % 2)

        if TAIL:
            def tail():
                pltpu.async_copy(tail_hbm, tv, sem_r[0])
                pltpu.make_async_copy(tail_hbm, tv, sem_r[0]).wait()

                def body(r, carry):
                    for j in range(E // _LANES):
                        rows0[pl.ds(r * E + j * _LANES, _LANES)] = (
                            tv[r, pl.ds(j * _LANES, _LANES)])
                    return carry
                lax.fori_loop(0, TAIL, body, 0)
                pltpu.async_copy(rows0.at[pl.ds(0, TAIL * E)],
                                 out_hbm.at[pl.ds(TC * SLOT, TAIL * E)],
                                 sem_w[0])
                pltpu.make_async_copy(rows0.at[pl.ds(0, TAIL * E)],
                                      out_hbm.at[pl.ds(TC * SLOT, TAIL * E)],
                                      sem_w[0]).wait()
            pl.when(wid == 0)(tail)

    return relayout_kernel(wtT, tail64)


def _gather_add(idsT, wt_flat, pos_flat, B, S, E):
    V = wt_flat.shape[0] // E
    wt2 = wt_flat.reshape(V, E)
    BL = 128
    NB = 8

    @functools.partial(
        pl.kernel,
        mesh=plsc.VectorSubcoreMesh(core_axis_name="c", subcore_axis_name="s"),
        out_type=jax.ShapeDtypeStruct((S, E // 8, B // BL, 8, BL),
                                      jnp.float32),
        compiler_params=pltpu.CompilerParams(
            use_tc_tiling_on_sc=False, needs_layout_passes=False),
        scratch_types=(
            [pltpu.VMEM((NB * BL,), jnp.int32)]
            + [pltpu.VMEM((BL, E), jnp.float32) for _ in range(NB)]
            + [pltpu.VMEM((E, BL), jnp.float32) for _ in range(2)]
            + [pltpu.VMEM((S * E,), jnp.float32)]
            + [pltpu.SemaphoreType.DMA] * (2 * NB + 3)
        ),
    )
    def gather_kernel(ids_hbm, wt_hbm, pos_hbm, out_hbm, idxr,
                      g0, g1, g2, g3, g4, g5, g6, g7, o0, o1, pos_v, *sems):
        gbuf = [g0, g1, g2, g3, g4, g5, g6, g7]
        obuf = [o0, o1]
        sem_i = list(sems[0:NB])
        sem_g = list(sems[NB:2 * NB])
        sem_w = list(sems[2 * NB:2 * NB + 2])
        sp = sems[2 * NB + 2]
        wid = lax.axis_index("s") * _NC + lax.axis_index("c")
        bbase = wid * BL

        pltpu.async_copy(pos_hbm, pos_v, sp).wait()

        def load_idx(s, b):
            pltpu.async_copy(ids_hbm.at[s, pl.ds(bbase, BL)],
                             idxr.at[pl.ds(b * BL, BL)], sem_i[b])

        def wait_idx(s, b):
            pltpu.make_async_copy(ids_hbm.at[s, pl.ds(bbase, BL)],
                                  idxr.at[pl.ds(b * BL, BL)],
                                  sem_i[b]).wait()

        def gather(s, b):
            pltpu.async_copy(wt_hbm.at[idxr.at[pl.ds(b * BL, BL)]],
                             gbuf[b], sem_g[b])

        def wait_gather(s, b):
            pltpu.make_async_copy(wt_hbm.at[idxr.at[pl.ds(b * BL, BL)]],
                                  gbuf[b], sem_g[b]).wait()

        def write(s, w):
            for e8 in range(E // 8):
                pltpu.async_copy(obuf[w].at[pl.ds(8 * e8, 8)],
                                 out_hbm.at[s, e8, wid], sem_w[w])

        def wait_write(s, w):
            for e8 in range(E // 8):
                pltpu.make_async_copy(obuf[w].at[pl.ds(8 * e8, 8)],
                                      out_hbm.at[s, e8, wid],
                                      sem_w[w]).wait()

        lanes = lax.iota(jnp.int32, _LANES)
        rot = [(lanes + k) % _LANES for k in range(_LANES)]

        def process(s, b, w):
            pbase = s * E
            for c0 in range(0, E, _LANES):
                cvecs = [rot[k] + c0 for k in range(_LANES)]
                pvs = [plsc.load_gather(pos_v, [cv + pbase]) for cv in cvecs]

                def body(i, carry):
                    row_idx = lanes + i * _LANES
                    vals = [plsc.load_gather(gbuf[b], [row_idx, cvecs[k]])
                            for k in range(_LANES)]
                    for k in range(_LANES):
                        plsc.store_scatter(obuf[w], [cvecs[k], row_idx],
                                           vals[k] + pvs[k])
                    return carry
                lax.fori_loop(0, BL // _LANES, body, 0)

        for b in range(NB):
            load_idx(b, b)
        for b in range(NB - 1):
            wait_idx(b, b)
            gather(b, b)

        def step(t, carry):
            for b in range(NB):
                s = t * NB + b
                w = b % 2
                def next_gather():
                    wait_idx(s + NB - 1, (b + NB - 1) % NB)
                    gather(s + NB - 1, (b + NB - 1) % NB)
                pl.when(s + NB - 1 < S)(next_gather)
                wait_gather(s, b)
                pl.when(s >= 2)(lambda: wait_write(s - 2, w))
                process(s, b, w)
                pl.when(s + NB < S)(lambda: load_idx(s + NB, b))
                write(s, w)
            return carry

        lax.fori_loop(0, S // NB, step, 0)
        for w in range(2):
            wait_write(S - 2 + w, (S - 2 + w) % 2)

    return gather_kernel(idsT, wt2, pos_flat)


def kernel(input_ids, word_table, pos_table):
    B, S = input_ids.shape
    V, E = word_table.shape

    wt_flat = _relayout(word_table)
    idsT = input_ids.T.astype(jnp.int32)
    pos_flat = pos_table.reshape(S * E)
    out5 = _gather_add(idsT, wt_flat, pos_flat, B, S, E)
    return out5.transpose(2, 4, 0, 1, 3).reshape(B, S, E)


# speedup vs baseline: 1.0467x; 1.0467x over previous
import functools

import jax
import jax.numpy as jnp
from jax import lax
from jax.experimental import pallas as pl
from jax.experimental.pallas import tpu as pltpu
from jax.experimental.pallas import tpu_sc as plsc

_LANES = 16
_NC, _NS = 2, 16
_NW = _NC * _NS


def _relayout(word_table):
    V, E = word_table.shape
    TC = V // 128
    TAIL = V - TC * 128
    base_per_w = TC // _NW
    rem = TC - base_per_w * _NW
    per_w = base_per_w + (1 if rem else 0)
    SLOT = 128 * E

    wtT = word_table.T
    tail64 = word_table[TC * 128:]

    @functools.partial(
        pl.kernel,
        mesh=plsc.VectorSubcoreMesh(core_axis_name="c", subcore_axis_name="s"),
        out_type=jax.ShapeDtypeStruct((V * E,), jnp.float32),
        compiler_params=pltpu.CompilerParams(
            use_tc_tiling_on_sc=True, needs_layout_passes=False),
        scratch_types=[
            pltpu.VMEM((E, 128), jnp.float32),
            pltpu.VMEM((E, 128), jnp.float32),
            pltpu.VMEM((128 * 64,), jnp.float32),
            pltpu.VMEM((128 * 64,), jnp.float32),
            pltpu.VMEM((64, 64), jnp.float32),
            pltpu.SemaphoreType.DMA,
            pltpu.SemaphoreType.DMA,
            pltpu.SemaphoreType.DMA,
            pltpu.SemaphoreType.DMA,
        ],
    )
    def relayout_kernel(wtT_hbm, tail_hbm, out_hbm, tiles0, tiles1,
                        rows0, rows1, tv, sr0, sr1, sw0, sw1):
        sem_r = [sr0, sr1]
        sem_w = [sw0, sw1]
        tiles = [tiles0, tiles1]
        rows = [rows0, rows1]
        wid = lax.axis_index("s") * _NC + lax.axis_index("c")
        start = wid * base_per_w + jnp.minimum(wid, rem)
        n_slots = per_w

        def slot(i):
            return jnp.minimum(start + i, TC - 1)

        def read(i, b):
            for c8 in range(E // 8):
                pltpu.async_copy(
                    wtT_hbm.at[pl.ds(8 * c8, 8), pl.ds(slot(i) * 128, 128)],
                    tiles[b].at[pl.ds(8 * c8, 8)], sem_r[b])

        def wait_read(i, b):
            for c8 in range(E // 8):
                pltpu.make_async_copy(
                    wtT_hbm.at[pl.ds(8 * c8, 8), pl.ds(slot(i) * 128, 128)],
                    tiles[b].at[pl.ds(8 * c8, 8)], sem_r[b]).wait()

        def write(i, b):
            pltpu.async_copy(rows[b], out_hbm.at[pl.ds(slot(i) * SLOT, SLOT)],
                             sem_w[b])

        def wait_write(i, b):
            pltpu.make_async_copy(rows[b],
                                  out_hbm.at[pl.ds(slot(i) * SLOT, SLOT)],
                                  sem_w[b]).wait()

        lanes = lax.iota(jnp.int32, _LANES)
        rot = [(lanes + k) % _LANES for k in range(_LANES)]
        d64 = [lanes * E + rot[k] for k in range(_LANES)]

        def transpose(b):
            def body(i, carry):
                row_idx = lanes + i * _LANES
                for c0 in range(0, E, _LANES):
                    dbase = i * (_LANES * E) + c0
                    vals = [plsc.load_gather(tiles[b], [rot[k] + c0, row_idx])
                            for k in range(_LANES)]
                    for k in range(_LANES):
                        plsc.store_scatter(rows[b], [d64[k] + dbase], vals[k])
                return carry
            lax.fori_loop(0, 128 // _LANES, body, 0)

        read(0, 0)

        def step(s, carry):
            for b in range(2):
                i = s * 2 + b
                pl.when(i + 1 < n_slots)(lambda: read(i + 1, 1 - b))
                wait_read(i, b)
                pl.when(i >= 2)(lambda: wait_write(i - 2, b))
                transpose(b)
                write(i, b)
            return carry

        lax.fori_loop(0, n_slots // 2, step, 0)
        if n_slots % 2:
            i = n_slots - 1
            b = i % 2
            wait_read(i, b)
            pl.when(i >= 2)(lambda: wait_write(i - 2, b))
            transpose(b)
            write(i, b)
        for b in range(2):
            i = n_slots - 2 + b
            wait_write(i, i % 2)

        if TAIL:
            def tail():
                pltpu.async_copy(tail_hbm, tv, sem_r[0])
                pltpu.make_async_copy(tail_hbm, tv, sem_r[0]).wait()

                def body(r, carry):
                    for j in range(E // _LANES):
                        rows0[pl.ds(r * E + j * _LANES, _LANES)] = (
                            tv[r, pl.ds(j * _LANES, _LANES)])
                    return carry
                lax.fori_loop(0, TAIL, body, 0)
                pltpu.async_copy(rows0.at[pl.ds(0, TAIL * E)],
                                 out_hbm.at[pl.ds(TC * SLOT, TAIL * E)],
                                 sem_w[0])
                pltpu.make_async_copy(rows0.at[pl.ds(0, TAIL * E)],
                                      out_hbm.at[pl.ds(TC * SLOT, TAIL * E)],
                                      sem_w[0]).wait()
            pl.when(wid == 0)(tail)

    return relayout_kernel(wtT, tail64)


def _gather_add(idsT, wt_flat, pos_flat, B, S, E):
    V = wt_flat.shape[0] // E
    wt2 = wt_flat.reshape(V, E)
    BL = 128
    NB = 4

    @functools.partial(
        pl.kernel,
        mesh=plsc.VectorSubcoreMesh(core_axis_name="c", subcore_axis_name="s"),
        out_type=jax.ShapeDtypeStruct((S, E // 8, B // BL, 8, BL),
                                      jnp.float32),
        compiler_params=pltpu.CompilerParams(
            use_tc_tiling_on_sc=False, needs_layout_passes=False),
        scratch_types=(
            [pltpu.VMEM((NB * BL,), jnp.int32)]
            + [pltpu.VMEM((BL, E), jnp.float32) for _ in range(NB)]
            + [pltpu.VMEM((E, BL), jnp.float32) for _ in range(2)]
            + [pltpu.VMEM((S * E,), jnp.float32)]
            + [pltpu.SemaphoreType.DMA] * (2 * NB + 3)
        ),
    )
    def gather_kernel(ids_hbm, wt_hbm, pos_hbm, out_hbm, idxr,
                      g0, g1, g2, g3, o0, o1, pos_v, *sems):
        gbuf = [g0, g1, g2, g3]
        obuf = [o0, o1]
        sem_i = list(sems[0:NB])
        sem_g = list(sems[NB:2 * NB])
        sem_w = list(sems[2 * NB:2 * NB + 2])
        sp = sems[2 * NB + 2]
        wid = lax.axis_index("s") * _NC + lax.axis_index("c")
        bbase = wid * BL

        pltpu.async_copy(pos_hbm, pos_v, sp).wait()

        def load_idx(s, b):
            pltpu.async_copy(ids_hbm.at[s, pl.ds(bbase, BL)],
                             idxr.at[pl.ds(b * BL, BL)], sem_i[b])

        def wait_idx(s, b):
            pltpu.make_async_copy(ids_hbm.at[s, pl.ds(bbase, BL)],
                                  idxr.at[pl.ds(b * BL, BL)],
                                  sem_i[b]).wait()

        def gather(s, b):
            pltpu.async_copy(wt_hbm.at[idxr.at[pl.ds(b * BL, BL)]],
                             gbuf[b], sem_g[b])

        def wait_gather(s, b):
            pltpu.make_async_copy(wt_hbm.at[idxr.at[pl.ds(b * BL, BL)]],
                                  gbuf[b], sem_g[b]).wait()

        def write(s, w):
            for e8 in range(E // 8):
                pltpu.async_copy(obuf[w].at[pl.ds(8 * e8, 8)],
                                 out_hbm.at[s, e8, wid], sem_w[w])

        def wait_write(s, w):
            for e8 in range(E // 8):
                pltpu.make_async_copy(obuf[w].at[pl.ds(8 * e8, 8)],
                                      out_hbm.at[s, e8, wid],
                                      sem_w[w]).wait()

        lanes = lax.iota(jnp.int32, _LANES)
        rot = [(lanes + k) % _LANES for k in range(_LANES)]

        def process(s, b, w):
            pbase = s * E
            for c0 in range(0, E, _LANES):
                cvecs = [rot[k] + c0 for k in range(_LANES)]
                pvs = [plsc.load_gather(pos_v, [cv + pbase]) for cv in cvecs]

                def body(i, carry):
                    row_idx = lanes + i * _LANES
                    vals = [plsc.load_gather(gbuf[b], [row_idx, cvecs[k]])
                            for k in range(_LANES)]
                    for k in range(_LANES):
                        plsc.store_scatter(obuf[w], [cvecs[k], row_idx],
                                           vals[k] + pvs[k])
                    return carry
                lax.fori_loop(0, BL // _LANES, body, 0)

        for b in range(NB):
            load_idx(b, b)
        for b in range(NB - 1):
            wait_idx(b, b)
            gather(b, b)

        def step(t, carry):
            for b in range(NB):
                s = t * NB + b
                w = b % 2
                def next_gather():
                    wait_idx(s + NB - 1, (b + NB - 1) % NB)
                    gather(s + NB - 1, (b + NB - 1) % NB)
                pl.when(s + NB - 1 < S)(next_gather)
                wait_gather(s, b)
                pl.when(s >= 2)(lambda: wait_write(s - 2, w))
                process(s, b, w)
                pl.when(s + NB < S)(lambda: load_idx(s + NB, b))
                write(s, w)
            return carry

        lax.fori_loop(0, S // NB, step, 0)
        for w in range(2):
            wait_write(S - 2 + w, (S - 2 + w) % 2)

    return gather_kernel(idsT, wt2, pos_flat)


def kernel(input_ids, word_table, pos_table):
    B, S = input_ids.shape
    V, E = word_table.shape

    wt_flat = _relayout(word_table)
    idsT = input_ids.T.astype(jnp.int32)
    pos_flat = pos_table.reshape(S * E)
    out5 = _gather_add(idsT, wt_flat, pos_flat, B, S, E)
    return out5.transpose(2, 4, 0, 1, 3).reshape(B, S, E)


# speedup vs baseline: 1.0516x; 1.0047x over previous
import functools

import jax
import jax.numpy as jnp
from jax import lax
from jax.experimental import pallas as pl
from jax.experimental.pallas import tpu as pltpu
from jax.experimental.pallas import tpu_sc as plsc

_LANES = 16
_NC, _NS = 2, 16
_NW = _NC * _NS


def _relayout(word_table):
    V, E = word_table.shape
    TC = V // 128
    TAIL = V - TC * 128
    base_per_w = TC // _NW
    rem = TC - base_per_w * _NW
    per_w = base_per_w + (1 if rem else 0)
    SLOT = 128 * E

    wtT = word_table.T
    tail64 = word_table[TC * 128:]

    @functools.partial(
        pl.kernel,
        mesh=plsc.VectorSubcoreMesh(core_axis_name="c", subcore_axis_name="s"),
        out_type=jax.ShapeDtypeStruct((V * E,), jnp.float32),
        compiler_params=pltpu.CompilerParams(
            use_tc_tiling_on_sc=True, needs_layout_passes=False),
        scratch_types=[
            pltpu.VMEM((E, 128), jnp.float32),
            pltpu.VMEM((E, 128), jnp.float32),
            pltpu.VMEM((128 * 64,), jnp.float32),
            pltpu.VMEM((128 * 64,), jnp.float32),
            pltpu.VMEM((64, 64), jnp.float32),
            pltpu.SemaphoreType.DMA,
            pltpu.SemaphoreType.DMA,
            pltpu.SemaphoreType.DMA,
            pltpu.SemaphoreType.DMA,
        ],
    )
    def relayout_kernel(wtT_hbm, tail_hbm, out_hbm, tiles0, tiles1,
                        rows0, rows1, tv, sr0, sr1, sw0, sw1):
        sem_r = [sr0, sr1]
        sem_w = [sw0, sw1]
        tiles = [tiles0, tiles1]
        rows = [rows0, rows1]
        wid = lax.axis_index("s") * _NC + lax.axis_index("c")
        start = wid * base_per_w + jnp.minimum(wid, rem)
        n_slots = per_w

        def slot(i):
            return jnp.minimum(start + i, TC - 1)

        def read(i, b):
            for c8 in range(E // 8):
                pltpu.async_copy(
                    wtT_hbm.at[pl.ds(8 * c8, 8), pl.ds(slot(i) * 128, 128)],
                    tiles[b].at[pl.ds(8 * c8, 8)], sem_r[b])

        def wait_read(i, b):
            for c8 in range(E // 8):
                pltpu.make_async_copy(
                    wtT_hbm.at[pl.ds(8 * c8, 8), pl.ds(slot(i) * 128, 128)],
                    tiles[b].at[pl.ds(8 * c8, 8)], sem_r[b]).wait()

        def write(i, b):
            pltpu.async_copy(rows[b], out_hbm.at[pl.ds(slot(i) * SLOT, SLOT)],
                             sem_w[b])

        def wait_write(i, b):
            pltpu.make_async_copy(rows[b],
                                  out_hbm.at[pl.ds(slot(i) * SLOT, SLOT)],
                                  sem_w[b]).wait()

        lanes = lax.iota(jnp.int32, _LANES)
        rot = [(lanes + k) % _LANES for k in range(_LANES)]
        d64 = [lanes * E + rot[k] for k in range(_LANES)]

        def transpose(b):
            def body(i, carry):
                row_idx = lanes + i * _LANES
                for c0 in range(0, E, _LANES):
                    dbase = i * (_LANES * E) + c0
                    vals = [plsc.load_gather(tiles[b], [rot[k] + c0, row_idx])
                            for k in range(_LANES)]
                    for k in range(_LANES):
                        plsc.store_scatter(rows[b], [d64[k] + dbase], vals[k])
                return carry
            lax.fori_loop(0, 128 // _LANES, body, 0)

        read(0, 0)

        def step(s, carry):
            for b in range(2):
                i = s * 2 + b
                pl.when(i + 1 < n_slots)(lambda: read(i + 1, 1 - b))
                wait_read(i, b)
                pl.when(i >= 2)(lambda: wait_write(i - 2, b))
                transpose(b)
                write(i, b)
            return carry

        lax.fori_loop(0, n_slots // 2, step, 0)
        if n_slots % 2:
            i = n_slots - 1
            b = i % 2
            wait_read(i, b)
            pl.when(i >= 2)(lambda: wait_write(i - 2, b))
            transpose(b)
            write(i, b)
        for b in range(2):
            i = n_slots - 2 + b
            wait_write(i, i % 2)

        if TAIL:
            def tail():
                pltpu.async_copy(tail_hbm, tv, sem_r[0])
                pltpu.make_async_copy(tail_hbm, tv, sem_r[0]).wait()

                def body(r, carry):
                    for j in range(E // _LANES):
                        rows0[pl.ds(r * E + j * _LANES, _LANES)] = (
                            tv[r, pl.ds(j * _LANES, _LANES)])
                    return carry
                lax.fori_loop(0, TAIL, body, 0)
                pltpu.async_copy(rows0.at[pl.ds(0, TAIL * E)],
                                 out_hbm.at[pl.ds(TC * SLOT, TAIL * E)],
                                 sem_w[0])
                pltpu.make_async_copy(rows0.at[pl.ds(0, TAIL * E)],
                                      out_hbm.at[pl.ds(TC * SLOT, TAIL * E)],
                                      sem_w[0]).wait()
            pl.when(wid == 0)(tail)

    return relayout_kernel(wtT, tail64)


def _gather_add(idsT, wt_flat, pos_flat, B, S, E):
    V = wt_flat.shape[0] // E
    wt2 = wt_flat.reshape(V, E)
    BL = 128
    NB = 4

    @functools.partial(
        pl.kernel,
        mesh=plsc.VectorSubcoreMesh(core_axis_name="c", subcore_axis_name="s"),
        out_type=jax.ShapeDtypeStruct((S, E // 8, B // BL, 8, BL),
                                      jnp.float32),
        compiler_params=pltpu.CompilerParams(
            use_tc_tiling_on_sc=False, needs_layout_passes=False),
        scratch_types=(
            [pltpu.VMEM((NB * BL,), jnp.int32)]
            + [pltpu.VMEM((BL, E), jnp.float32) for _ in range(NB)]
            + [pltpu.VMEM((E, BL), jnp.float32) for _ in range(4)]
            + [pltpu.VMEM((S * E,), jnp.float32)]
            + [pltpu.SemaphoreType.DMA] * (2 * NB + 5)
        ),
    )
    def gather_kernel(ids_hbm, wt_hbm, pos_hbm, out_hbm, idxr,
                      g0, g1, g2, g3, o0, o1, o2, o3, pos_v, *sems):
        gbuf = [g0, g1, g2, g3]
        obuf = [o0, o1, o2, o3]
        sem_i = list(sems[0:NB])
        sem_g = list(sems[NB:2 * NB])
        sem_w = list(sems[2 * NB:2 * NB + 4])
        sp = sems[2 * NB + 4]
        wid = lax.axis_index("s") * _NC + lax.axis_index("c")
        bbase = wid * BL

        pltpu.async_copy(pos_hbm, pos_v, sp).wait()

        def load_idx(s, b):
            pltpu.async_copy(ids_hbm.at[s, pl.ds(bbase, BL)],
                             idxr.at[pl.ds(b * BL, BL)], sem_i[b])

        def wait_idx(s, b):
            pltpu.make_async_copy(ids_hbm.at[s, pl.ds(bbase, BL)],
                                  idxr.at[pl.ds(b * BL, BL)],
                                  sem_i[b]).wait()

        def gather(s, b):
            pltpu.async_copy(wt_hbm.at[idxr.at[pl.ds(b * BL, BL)]],
                             gbuf[b], sem_g[b])

        def wait_gather(s, b):
            pltpu.make_async_copy(wt_hbm.at[idxr.at[pl.ds(b * BL, BL)]],
                                  gbuf[b], sem_g[b]).wait()

        def write(s, w):
            for e8 in range(E // 8):
                pltpu.async_copy(obuf[w].at[pl.ds(8 * e8, 8)],
                                 out_hbm.at[s, e8, wid], sem_w[w])

        def wait_write(s, w):
            for e8 in range(E // 8):
                pltpu.make_async_copy(obuf[w].at[pl.ds(8 * e8, 8)],
                                      out_hbm.at[s, e8, wid],
                                      sem_w[w]).wait()

        lanes = lax.iota(jnp.int32, _LANES)
        rot = [(lanes + k) % _LANES for k in range(_LANES)]

        def process(s, b, w):
            pbase = s * E
            for c0 in range(0, E, _LANES):
                cvecs = [rot[k] + c0 for k in range(_LANES)]
                pvs = [plsc.load_gather(pos_v, [cv + pbase]) for cv in cvecs]

                def body(i, carry):
                    row_idx = lanes + i * _LANES
                    vals = [plsc.load_gather(gbuf[b], [row_idx, cvecs[k]])
                            for k in range(_LANES)]
                    for k in range(_LANES):
                        plsc.store_scatter(obuf[w], [cvecs[k], row_idx],
                                           vals[k] + pvs[k])
                    return carry
                lax.fori_loop(0, BL // _LANES, body, 0)

        for b in range(NB):
            load_idx(b, b)
        for b in range(NB - 1):
            wait_idx(b, b)
            gather(b, b)

        def step(t, carry):
            for b in range(NB):
                s = t * NB + b
                w = b
                def next_gather():
                    wait_idx(s + NB - 1, (b + NB - 1) % NB)
                    gather(s + NB - 1, (b + NB - 1) % NB)
                pl.when(s + NB - 1 < S)(next_gather)
                wait_gather(s, b)
                pl.when(s >= 4)(lambda: wait_write(s - 4, w))
                process(s, b, w)
                pl.when(s + NB < S)(lambda: load_idx(s + NB, b))
                write(s, w)
            return carry

        lax.fori_loop(0, S // NB, step, 0)
        for w in range(4):
            wait_write(S - 4 + w, (S - 4 + w) % 4)

    return gather_kernel(idsT, wt2, pos_flat)


def kernel(input_ids, word_table, pos_table):
    B, S = input_ids.shape
    V, E = word_table.shape

    wt_flat = _relayout(word_table)
    idsT = input_ids.T.astype(jnp.int32)
    pos_flat = pos_table.reshape(S * E)
    out5 = _gather_add(idsT, wt_flat, pos_flat, B, S, E)
    return out5.transpose(2, 4, 0, 1, 3).reshape(B, S, E)
